# baseline (speedup 1.0000x reference)
import functools

import jax
import jax.numpy as jnp
from jax import lax
from jax.experimental import pallas as pl
from jax.experimental.pallas import tpu as pltpu
from jax.experimental.pallas import tpu_sc as plsc

M = N = 4096
BS = 16
NB = M // BS
K = int(int(M * N * 0.05) / (BS * BS))
GRID = 16
ROWS = M // GRID
G1 = 16
W = M // G1
WB = W // BS


def _sel_matrix(rows, cols, fn):
    i = lax.broadcasted_iota(jnp.int32, (rows, cols), 0)
    j = lax.broadcasted_iota(jnp.int32, (rows, cols), 1)
    return jnp.where(fn(i, j), 1.0, 0.0).astype(jnp.float32)


def _dot(a, b, precision=lax.Precision.HIGHEST):
    return lax.dot_general(a, b, (((1,), (0,)), ((), ())),
                           precision=precision,
                           preferred_element_type=jnp.float32)


CHW = 256


def _scores_kernel(grad_ref, out_ref):
    x = grad_ref[...]
    for c in range(W // CHW):
        xc = x[:, c * CHW:(c + 1) * CHW]
        x5 = xc.reshape(NB, BS, CHW)
        acc = x5[:, 0, :]
        for r in range(1, BS):
            acc = acc + x5[:, r, :]
        accT = lax.transpose(acc, (1, 0))
        x6 = accT.reshape(CHW // BS, BS, NB)
        t = x6[:, 0:8, :] + x6[:, 8:16, :]
        t = t[:, 0:4, :] + t[:, 4:8, :]
        t = t[:, 0:2, :] + t[:, 2:4, :]
        t = t[:, 0:1, :] + t[:, 1:2, :]
        nb = CHW // BS
        out_ref[c * nb:(c + 1) * nb, :] = t.reshape(nb, NB)


NE = NB * NB
PER = NE // 16


def _sc_select(scores_flat):
    @functools.partial(
        pl.kernel,
        out_type=jax.ShapeDtypeStruct((16,), jnp.int32),
        mesh=plsc.VectorSubcoreMesh(core_axis_name="c", subcore_axis_name="s"),
        scratch_types=[
            pltpu.VMEM((PER,), jnp.int32),
            pltpu.VMEM((16,), jnp.float32),
            pltpu.VMEM((16,), jnp.float32),
            pltpu.VMEM((16,), jnp.int32),
            pltpu.VMEM((16,), jnp.int32),
            pltpu.VMEM((1024,), jnp.float32),
            pltpu.VMEM_SHARED((1024,), jnp.float32),
        ],
    )
    def k(sc_hbm, out_hbm, xv, cntv, rbv, idxv, outv, zv, shh):
        c = lax.axis_index("c")
        s = lax.axis_index("s")
        zeros16 = jnp.zeros((16,), jnp.float32)
        ones16 = jnp.ones((16,), jnp.float32)
        lane = lax.broadcasted_iota(jnp.int32, (16,), 0)

        pltpu.sync_copy(sc_hbm.at[pl.ds(s * PER, PER)], xv)

        @pl.when(s == 0)
        def _():
            for q in range(64):
                zv[pl.ds(q * 16, 16)] = zeros16
            pltpu.sync_copy(zv, shh)

        plsc.subcore_barrier()

        def merged_count(cand, slot):
            cand16 = jnp.broadcast_to(cand, (16,))

            def body(i, acc):
                m = xv[pl.ds(i * 16, 16)]
                return acc + lax.select(m >= cand16, ones16, zeros16)

            local = lax.fori_loop(0, PER // 16, body, zeros16)
            cntv[...] = local
            idxv[...] = lane + slot * 16
            pltpu.sync_copy(cntv, shh.at[idxv], add=True)
            plsc.subcore_barrier()
            pltpu.sync_copy(shh.at[pl.ds(slot * 16, 16)], rbv)
            v = rbv[...]
            tot = v[0]
            for q in range(1, 16):
                tot = tot + v[q]
            return tot

        kf = jnp.float32(K)
        cnt_pos = merged_count(jnp.int32(0), 0)
        cur = jnp.where(cnt_pos >= kf, jnp.int32(0), jnp.int32(-2147483648))
        for rnd in range(31):
            cand = cur + jnp.int32(1 << (30 - rnd))
            cnt = merged_count(cand, 1 + rnd)
            cur = jnp.where(cnt >= kf, cand, cur)
        count_gt = merged_count(cur + jnp.int32(1), 32)

        ovec = lax.select(lane == 0, jnp.broadcast_to(cur, (16,)),
                          jnp.broadcast_to(count_gt.astype(jnp.int32), (16,)))
        ovec = lax.select(lane <= 1, ovec, jnp.zeros((16,), jnp.int32))

        @pl.when((c == 0) & (s == 0))
        def _():
            outv[...] = ovec
            pltpu.sync_copy(outv, out_hbm)

    return k(scores_flat)


def _mask_expand_kernel(scT_ref, sel_ref, out_ref, mask):
    i = pl.program_id(0)

    @pl.when(i == 0)
    def _():
        sc = scT_ref[...]
        b = lax.bitcast_convert_type(sc, jnp.int32)
        mag = b & jnp.int32(0x7FFFFFFF)
        m = jnp.where(b >= 0, b, jnp.int32(-1) - mag)
        selv = sel_ref[...]
        T = selv[0, 0]
        gt = m > T
        eq = m == T
        need = jnp.float32(K) - selv[0, 1].astype(jnp.float32)
        eq_f = eq.astype(jnp.float32)
        s = _dot(jnp.ones((1, NB), jnp.float32), eq_f,
                 lax.Precision.DEFAULT)
        Um = _sel_matrix(NB, NB, lambda a, b2: a < b2)
        w1 = _dot(s, Um, lax.Precision.DEFAULT)
        Lm = _sel_matrix(NB, NB, lambda a, b2: a > b2)
        W2 = _dot(Lm, eq_f, lax.Precision.DEFAULT)
        prefix = w1 + W2
        tie = eq & (prefix < need)
        maskT = jnp.where(gt | tie, 1.0, 0.0).astype(jnp.float32)
        I = _sel_matrix(NB, NB, lambda a, b2: a == b2)
        mask[...] = lax.dot_general(maskT, I, (((0,), (0,)), ((), ())),
                                    precision=lax.Precision.HIGHEST,
                                    preferred_element_type=jnp.float32)

    sub = mask[pl.ds(pl.multiple_of(i * BS, BS), BS), :]
    E = _sel_matrix(NB, N, lambda b2, j: b2 == j // BS)
    ex = _dot(sub, E, lax.Precision.DEFAULT)
    R = _sel_matrix(ROWS, BS, lambda r, c: r // BS == c)
    out_ref[...] = _dot(R, ex, lax.Precision.DEFAULT)


def kernel(weight, grad):
    scT = pl.pallas_call(
        _scores_kernel,
        grid=(G1,),
        in_specs=[pl.BlockSpec((M, W), lambda i: (0, i))],
        out_specs=pl.BlockSpec((WB, NB), lambda i: (i, 0)),
        out_shape=jax.ShapeDtypeStruct((NB, NB), jnp.float32),
        compiler_params=pltpu.CompilerParams(
            dimension_semantics=("arbitrary",)),
    )(grad)
    bb = lax.bitcast_convert_type(scT.reshape(NB * NB), jnp.int32)
    menc = jnp.where(bb >= 0, bb,
                     jnp.int32(-1) - (bb & jnp.int32(0x7FFFFFFF)))
    sel = _sc_select(menc).reshape(1, 16)
    out = pl.pallas_call(
        _mask_expand_kernel,
        grid=(GRID,),
        in_specs=[pl.BlockSpec((NB, NB), lambda i: (0, 0)),
                  pl.BlockSpec((1, 16), lambda i: (0, 0))],
        out_specs=pl.BlockSpec((ROWS, N), lambda i: (i, 0)),
        out_shape=jax.ShapeDtypeStruct((M, N), jnp.float32),
        scratch_shapes=[pltpu.VMEM((NB, NB), jnp.float32)],
        compiler_params=pltpu.CompilerParams(
            dimension_semantics=("arbitrary",)),
    )(scT, sel)
    return out.astype(weight.dtype)


# speedup vs baseline: 1.7778x; 1.7778x over previous
import jax
import jax.numpy as jnp
from jax import lax
from jax.experimental import pallas as pl
from jax.experimental.pallas import tpu as pltpu

M = N = 4096
BS = 16
NB = M // BS
K = int(int(M * N * 0.05) / (BS * BS))
GRID = 16
ROWS = M // GRID
G1 = 16
W = M // G1
WB = W // BS


def _sel_matrix(rows, cols, fn):
    i = lax.broadcasted_iota(jnp.int32, (rows, cols), 0)
    j = lax.broadcasted_iota(jnp.int32, (rows, cols), 1)
    return jnp.where(fn(i, j), 1.0, 0.0).astype(jnp.float32)


def _dot(a, b, precision=lax.Precision.HIGHEST):
    return lax.dot_general(a, b, (((1,), (0,)), ((), ())),
                           precision=precision,
                           preferred_element_type=jnp.float32)


CHW = 256


def _scores_kernel(grad_ref, out_ref):
    x = grad_ref[...]
    for c in range(W // CHW):
        xc = x[:, c * CHW:(c + 1) * CHW]
        x5 = xc.reshape(NB, BS, CHW)
        acc = x5[:, 0, :]
        for r in range(1, BS):
            acc = acc + x5[:, r, :]
        accT = lax.transpose(acc, (1, 0))
        x6 = accT.reshape(CHW // BS, BS, NB)
        t = x6[:, 0:8, :] + x6[:, 8:16, :]
        t = t[:, 0:4, :] + t[:, 4:8, :]
        t = t[:, 0:2, :] + t[:, 2:4, :]
        t = t[:, 0:1, :] + t[:, 1:2, :]
        nb = CHW // BS
        out_ref[c * nb:(c + 1) * nb, :] = t.reshape(nb, NB)


def _mask_expand_kernel(scT_ref, out_ref, mask):
    i = pl.program_id(0)

    @pl.when(i == 0)
    def _():
        sc = scT_ref[...]
        b = lax.bitcast_convert_type(sc, jnp.int32)
        mag = b & jnp.int32(0x7FFFFFFF)
        m = jnp.where(b >= 0, b, jnp.int32(-1) - mag)
        cnt_pos = jnp.sum((m >= 0).astype(jnp.int32))
        cur0 = jnp.where(cnt_pos >= K, jnp.int32(0), jnp.int32(-2147483648))

        def body(t, cur):
            cand = cur + (jnp.int32(1) << (30 - t))
            cnt = jnp.sum((m >= cand).astype(jnp.int32))
            return jnp.where(cnt >= K, cand, cur)

        T = lax.fori_loop(0, 31, body, cur0)
        gt = m > T
        eq = m == T
        need = jnp.float32(K) - jnp.sum(gt.astype(jnp.float32))
        eq_f = eq.astype(jnp.float32)
        s = _dot(jnp.ones((1, NB), jnp.float32), eq_f,
                 lax.Precision.DEFAULT)
        Um = _sel_matrix(NB, NB, lambda a, b2: a < b2)
        w1 = _dot(s, Um, lax.Precision.DEFAULT)
        Lm = _sel_matrix(NB, NB, lambda a, b2: a > b2)
        W2 = _dot(Lm, eq_f, lax.Precision.DEFAULT)
        prefix = w1 + W2
        tie = eq & (prefix < need)
        maskT = jnp.where(gt | tie, 1.0, 0.0).astype(jnp.float32)
        I = _sel_matrix(NB, NB, lambda a, b2: a == b2)
        mask[...] = lax.dot_general(maskT, I, (((0,), (0,)), ((), ())),
                                    precision=lax.Precision.HIGHEST,
                                    preferred_element_type=jnp.float32)

    sub = mask[pl.ds(pl.multiple_of(i * BS, BS), BS), :]
    E = _sel_matrix(NB, N, lambda b2, j: b2 == j // BS)
    ex = _dot(sub, E, lax.Precision.DEFAULT)
    R = _sel_matrix(ROWS, BS, lambda r, c: r // BS == c)
    out_ref[...] = _dot(R, ex, lax.Precision.DEFAULT)


def kernel(weight, grad):
    scT = pl.pallas_call(
        _scores_kernel,
        grid=(G1,),
        in_specs=[pl.BlockSpec((M, W), lambda i: (0, i))],
        out_specs=pl.BlockSpec((WB, NB), lambda i: (i, 0)),
        out_shape=jax.ShapeDtypeStruct((NB, NB), jnp.float32),
        compiler_params=pltpu.CompilerParams(
            dimension_semantics=("arbitrary",)),
    )(grad)
    out = pl.pallas_call(
        _mask_expand_kernel,
        grid=(GRID,),
        in_specs=[pl.BlockSpec((NB, NB), lambda i: (0, 0))],
        out_specs=pl.BlockSpec((ROWS, N), lambda i: (i, 0)),
        out_shape=jax.ShapeDtypeStruct((M, N), jnp.float32),
        scratch_shapes=[pltpu.VMEM((NB, NB), jnp.float32)],
        compiler_params=pltpu.CompilerParams(
            dimension_semantics=("arbitrary",)),
    )(scT)
    return out.astype(weight.dtype)
